# Initial kernel scaffold; baseline (speedup 1.0000x reference)
import jax
import jax.numpy as jnp
from jax.experimental import pallas as pl


def kernel(node_features, edge_list, degrees, kernel, bias):
    raise NotImplementedError("write your pallas kernel here")


# speedup vs baseline: 9.4641x; 9.4641x over previous
import functools

import jax
import jax.numpy as jnp
from jax import lax
from jax.experimental import pallas as pl
from jax.experimental.pallas import tpu as pltpu
from jax.experimental.pallas import tpu_sc as plsc

N_NODES = 10000
N_EDGES = 160000
D_FEAT = 256
UNITS = 256
HALF = 128

NUM_CORES = 2
NUM_SUBCORES = 16
CHUNK = 128

NP = 10240
STRIPE = NP // NUM_SUBCORES

SIDES = 2 * N_EDGES
SIDES_PER_TILE = -(-SIDES // (NUM_SUBCORES * CHUNK)) * CHUNK
SIDES_PAD = SIDES_PER_TILE * NUM_SUBCORES
CHUNKS_PER_TILE = SIDES_PER_TILE // CHUNK

ROW_BLK = 1000
GRID = N_NODES // ROW_BLK


def _pre_body(x_ref, deg_ref, lo_ref, hi_ref):
    scale = lax.rsqrt(jnp.maximum(deg_ref[...], 1.0))
    xs = x_ref[...] * scale
    lo_ref[...] = xs[:, :HALF]
    hi_ref[...] = xs[:, HALF:]


def _tc_pre(x, deg2):
    return pl.pallas_call(
        _pre_body,
        grid=(GRID,),
        in_specs=[
            pl.BlockSpec((ROW_BLK, D_FEAT), lambda i: (i, 0)),
            pl.BlockSpec((ROW_BLK, 1), lambda i: (i, 0)),
        ],
        out_specs=[
            pl.BlockSpec((ROW_BLK, HALF), lambda i: (i, 0)),
            pl.BlockSpec((ROW_BLK, HALF), lambda i: (i, 0)),
        ],
        out_shape=[
            jax.ShapeDtypeStruct((NP, HALF), jnp.float32),
            jax.ShapeDtypeStruct((NP, HALF), jnp.float32),
        ],
    )(x, deg2)


def _sc_body(xs_lo, xs_hi, srcs, dsts, zeros, out, sidx, didx, rows, acc, sem):
    c = lax.axis_index("c")
    s = lax.axis_index("s")

    row0 = pl.multiple_of(s * STRIPE, STRIPE)
    pltpu.sync_copy(zeros.at[pl.ds(row0, STRIPE)], acc.at[pl.ds(row0, STRIPE)])
    plsc.subcore_barrier()

    base = s * SIDES_PER_TILE

    def chunk(i, carry):
        off = pl.multiple_of(base + i * CHUNK, CHUNK)
        pltpu.sync_copy(srcs.at[pl.ds(off, CHUNK)], sidx)
        pltpu.sync_copy(dsts.at[pl.ds(off, CHUNK)], didx)

        @pl.when(c == 0)
        def _():
            pltpu.async_copy(xs_lo.at[sidx], rows, sem).wait()

        @pl.when(c == 1)
        def _():
            pltpu.async_copy(xs_hi.at[sidx], rows, sem).wait()

        pltpu.sync_copy(rows, acc.at[didx], add=True)
        return carry

    lax.fori_loop(0, CHUNKS_PER_TILE, chunk, 0)
    plsc.subcore_barrier()

    out0 = pl.multiple_of(c * NP + s * STRIPE, STRIPE)
    pltpu.sync_copy(acc.at[pl.ds(row0, STRIPE)], out.at[pl.ds(out0, STRIPE)])


def _sc_scatter(xs_lo, xs_hi, srcs, dsts, zeros):
    mesh = plsc.VectorSubcoreMesh(core_axis_name="c", subcore_axis_name="s")
    fn = functools.partial(
        pl.kernel,
        mesh=mesh,
        out_type=jax.ShapeDtypeStruct((NUM_CORES * NP, HALF), jnp.float32),
        scratch_types=[
            pltpu.VMEM((CHUNK,), jnp.int32),
            pltpu.VMEM((CHUNK,), jnp.int32),
            pltpu.VMEM((CHUNK, HALF), jnp.float32),
            pltpu.VMEM_SHARED((NP, HALF), jnp.float32),
            pltpu.SemaphoreType.DMA,
        ],
    )(_sc_body)
    return fn(xs_lo, xs_hi, srcs, dsts, zeros)


def _post_body(tmp_ref, x_ref, deg_ref, w_ref, b_ref, o_ref):
    deg = jnp.maximum(deg_ref[...], 1.0)
    agg = jnp.concatenate([tmp_ref[0], tmp_ref[1]], axis=1)
    agg = agg * lax.rsqrt(deg) + x_ref[...] * (1.0 / deg)
    y = jnp.dot(agg, w_ref[...], preferred_element_type=jnp.float32)
    o_ref[...] = jnp.maximum(y + b_ref[...], 0.0)


def _tc_post(tmp3, x, deg2, w, b2):
    return pl.pallas_call(
        _post_body,
        grid=(GRID,),
        in_specs=[
            pl.BlockSpec((NUM_CORES, ROW_BLK, HALF), lambda i: (0, i, 0)),
            pl.BlockSpec((ROW_BLK, D_FEAT), lambda i: (i, 0)),
            pl.BlockSpec((ROW_BLK, 1), lambda i: (i, 0)),
            pl.BlockSpec((D_FEAT, UNITS), lambda i: (0, 0)),
            pl.BlockSpec((1, UNITS), lambda i: (0, 0)),
        ],
        out_specs=pl.BlockSpec((ROW_BLK, UNITS), lambda i: (i, 0)),
        out_shape=jax.ShapeDtypeStruct((N_NODES, UNITS), jnp.float32),
    )(tmp3, x, deg2, w, b2)


def kernel(node_features, edge_list, degrees, kernel, bias):
    x = node_features
    deg2 = degrees.reshape(N_NODES, 1)

    xs_lo, xs_hi = _tc_pre(x, deg2)

    pad = SIDES_PAD - SIDES
    srcs = jnp.concatenate(
        [edge_list[:, 1], edge_list[:, 0], jnp.zeros((pad,), jnp.int32)])
    dsts = jnp.concatenate(
        [edge_list[:, 0], edge_list[:, 1],
         jnp.full((pad,), N_NODES, jnp.int32)])
    zeros = jnp.zeros((NP, HALF), jnp.float32)

    tmp = _sc_scatter(xs_lo, xs_hi, srcs, dsts, zeros)
    tmp3 = tmp.reshape(NUM_CORES, NP, HALF)

    return _tc_post(tmp3, x, deg2, kernel, bias.reshape(1, UNITS))
